# baseline (speedup 1.0000x reference)
import math

import jax
import jax.numpy as jnp
from jax.experimental import pallas as pl

S = 2048
H = 768
NH = 12
NKV = 4
HD = 64
HALF = HD // 2
I = 768
E = 64
THETA = 150000.0
EPS = 1e-6
BLK = 256
GRID = S // BLK
REP = NH // NKV
Q_SIZE = NH * HD
KV_SIZE = NKV * HD

_NEG = -1e30
_NT = (((1,), (1,)), ((), ()))


def _split_halves(x, nheads):
    h1 = [x[:, h * HD:h * HD + HALF] for h in range(nheads)]
    h2 = [x[:, h * HD + HALF:(h + 1) * HD] for h in range(nheads)]
    return jnp.concatenate(h1 + h2, axis=1)


def _stage1_body(pos_ref, hid_ref, res_ref, w_ref, b_ref, ln_ref,
                 q_out, k_out, v_out, r1_out):
    x = hid_ref[...] + res_ref[...]
    r1_out[...] = x
    ms = jnp.mean(x * x, axis=1, keepdims=True)
    h = x * jax.lax.rsqrt(ms + EPS) * ln_ref[...]
    qkv = jax.lax.dot_general(
        h.astype(jnp.bfloat16), w_ref[...], _NT,
        preferred_element_type=jnp.float32) + b_ref[...]

    pos = pos_ref[...]
    jq = jax.lax.rem(jax.lax.broadcasted_iota(jnp.int32, (1, NH * HALF), 1),
                     HALF).astype(jnp.float32)
    inv_freq = jnp.exp(jq * (-math.log(THETA) / HALF))
    f = pos * inv_freq
    cos_q = jnp.cos(f)
    sin_q = jnp.sin(f)
    cos_k = cos_q[:, :NKV * HALF]
    sin_k = sin_q[:, :NKV * HALF]

    qh = _split_halves(qkv[:, :Q_SIZE], NH)
    kh = _split_halves(qkv[:, Q_SIZE:Q_SIZE + KV_SIZE], NKV)
    q1 = qh[:, :NH * HALF]
    q2 = qh[:, NH * HALF:]
    k1 = kh[:, :NKV * HALF]
    k2 = kh[:, NKV * HALF:]
    v = qkv[:, Q_SIZE + KV_SIZE:]

    scale = HD ** -0.5
    q_out[...] = (jnp.concatenate(
        [q1 * cos_q - q2 * sin_q, q2 * cos_q + q1 * sin_q],
        axis=1) * scale).astype(jnp.bfloat16)
    k_out[...] = jnp.concatenate(
        [k1 * cos_k - k2 * sin_k, k2 * cos_k + k1 * sin_k],
        axis=1).astype(jnp.bfloat16)
    v_out[...] = v.astype(jnp.bfloat16)


def _stage2_body(q_ref, k_ref, v_ref, r1_ref, wo_ref, bo_ref, ln2_ref,
                 wr_ref, br_ref, wgu_ref, bgu_ref, wd_ref, bd_ref,
                 out_ref, r2_out):
    i = pl.program_id(0)
    q0 = i * BLK
    R = REP * BLK

    row = jax.lax.rem(jax.lax.broadcasted_iota(jnp.int32, (R, 1), 0), BLK)
    col = jax.lax.broadcasted_iota(jnp.int32, (1, BLK), 1)
    qidx = q0 + row

    o_cols = []
    for g in range(NKV):
        qs = []
        for hh in range(REP):
            h = g * REP + hh
            qs.append(jnp.concatenate(
                [q_ref[:, h * HALF:(h + 1) * HALF],
                 q_ref[:, NH * HALF + h * HALF:NH * HALF + (h + 1) * HALF]],
                axis=1))
        q_g = jnp.concatenate(qs, axis=0)

        def chunk_body(j, carry):
            m, l, acc = carry
            k_c = jnp.concatenate(
                [k_ref[pl.ds(j * BLK, BLK), g * HALF:(g + 1) * HALF],
                 k_ref[pl.ds(j * BLK, BLK),
                       NKV * HALF + g * HALF:NKV * HALF + (g + 1) * HALF]],
                axis=1)
            v_c = v_ref[pl.ds(j * BLK, BLK), g * HD:(g + 1) * HD]
            s = jax.lax.dot_general(q_g, k_c, _NT,
                                    preferred_element_type=jnp.float32)
            s = jnp.where((j * BLK + col) <= qidx, s, _NEG)
            m_new = jnp.maximum(m, jnp.max(s, axis=1, keepdims=True))
            alpha = jnp.exp(m - m_new)
            p = jnp.exp(s - m_new)
            l_new = l * alpha + jnp.sum(p, axis=1, keepdims=True)
            acc_new = acc * alpha + jnp.dot(
                p.astype(jnp.bfloat16), v_c,
                preferred_element_type=jnp.float32)
            return (m_new, l_new, acc_new)

        m0 = jnp.full((R, 1), _NEG, jnp.float32)
        l0 = jnp.zeros((R, 1), jnp.float32)
        a0 = jnp.zeros((R, HD), jnp.float32)
        m, l, acc = jax.lax.fori_loop(0, i + 1, chunk_body, (m0, l0, a0))
        o_g = acc / l
        for hh in range(REP):
            o_cols.append(o_g[hh * BLK:(hh + 1) * BLK, :])
    o = jnp.concatenate(o_cols, axis=1).astype(jnp.bfloat16)

    attn = jax.lax.dot_general(
        o, wo_ref[...], _NT, preferred_element_type=jnp.float32) + bo_ref[...]
    r2 = attn + r1_ref[...]
    r2_out[...] = r2

    ms = jnp.mean(r2 * r2, axis=1, keepdims=True)
    h2 = (r2 * jax.lax.rsqrt(ms + EPS) * ln2_ref[...]).astype(jnp.bfloat16)

    logits = jax.lax.dot_general(
        h2, wr_ref[...], _NT, preferred_element_type=jnp.float32) + br_ref[...]
    m1 = jnp.max(logits, axis=1, keepdims=True)
    s2 = jnp.max(jnp.where(logits >= m1, _NEG, logits), axis=1, keepdims=True)
    e2 = jnp.exp(s2 - m1)
    denom = 1.0 + e2
    factor = 1.0 / denom + e2 / denom

    gu = jax.lax.dot_general(
        h2, wgu_ref[...], _NT,
        preferred_element_type=jnp.float32) + bgu_ref[...]
    gate = gu[:, :I]
    up = gu[:, I:]
    x = gate * (up * jax.nn.sigmoid(up))
    eo = jax.lax.dot_general(
        x.astype(jnp.bfloat16), wd_ref[...], _NT,
        preferred_element_type=jnp.float32) + bd_ref[...]
    out_ref[...] = factor * eo


def kernel(positions, hidden_states, residual, w_qkv, b_qkv, w_o, b_o,
           w_router, b_router, w_gate_up, b_gate_up, w_down, b_down,
           ln1_w, ln2_w):
    f32 = jnp.float32
    bf16 = jnp.bfloat16
    pos = positions.astype(f32).reshape(S, 1)

    full = lambda shape: pl.BlockSpec(shape, lambda i: (0, 0))
    blk = lambda cols: pl.BlockSpec((BLK, cols), lambda i: (i, 0))

    q_ro, k_ro, v, r1 = pl.pallas_call(
        _stage1_body,
        grid=(GRID,),
        in_specs=[
            blk(1),
            blk(H),
            blk(H),
            full((Q_SIZE + 2 * KV_SIZE, H)),
            full((1, Q_SIZE + 2 * KV_SIZE)),
            full((1, H)),
        ],
        out_specs=[blk(Q_SIZE), blk(KV_SIZE), blk(KV_SIZE), blk(H)],
        out_shape=[
            jax.ShapeDtypeStruct((S, Q_SIZE), bf16),
            jax.ShapeDtypeStruct((S, KV_SIZE), bf16),
            jax.ShapeDtypeStruct((S, KV_SIZE), bf16),
            jax.ShapeDtypeStruct((S, H), f32),
        ],
    )(pos, hidden_states, residual, w_qkv.astype(bf16),
      b_qkv.reshape(1, -1).astype(f32), ln1_w.reshape(1, H).astype(f32))

    out, r2 = pl.pallas_call(
        _stage2_body,
        grid=(GRID,),
        in_specs=[
            blk(Q_SIZE),
            full((S, KV_SIZE)),
            full((S, KV_SIZE)),
            blk(H),
            full((H, Q_SIZE)),
            full((1, H)),
            full((1, H)),
            full((E, H)),
            full((1, E)),
            full((2 * I, H)),
            full((1, 2 * I)),
            full((H, I)),
            full((1, H)),
        ],
        out_specs=[blk(H), blk(H)],
        out_shape=[
            jax.ShapeDtypeStruct((S, H), f32),
            jax.ShapeDtypeStruct((S, H), f32),
        ],
    )(q_ro, k_ro, v, r1,
      w_o.astype(bf16), b_o.reshape(1, H).astype(f32),
      ln2_w.reshape(1, H).astype(f32),
      w_router.astype(bf16), b_router.reshape(1, E).astype(f32),
      w_gate_up.astype(bf16), b_gate_up.reshape(1, 2 * I).astype(f32),
      w_down.astype(bf16), b_down.reshape(1, H).astype(f32))

    return (out, r2)


# speedup vs baseline: 1.2562x; 1.2562x over previous
import math

import jax
import jax.numpy as jnp
from jax.experimental import pallas as pl

S = 2048
H = 768
NH = 12
NKV = 4
HD = 64
HALF = HD // 2
I = 768
E = 64
THETA = 150000.0
EPS = 1e-6
BLK = 256
GRID = S // BLK
REP = NH // NKV
Q_SIZE = NH * HD
KV_SIZE = NKV * HD

_NEG = -1e30
_NT = (((1,), (1,)), ((), ()))


def _split_halves(x, nheads):
    h1 = [x[:, h * HD:h * HD + HALF] for h in range(nheads)]
    h2 = [x[:, h * HD + HALF:(h + 1) * HD] for h in range(nheads)]
    return jnp.concatenate(h1 + h2, axis=1)


def _stage1_body(pos_ref, hid_ref, res_ref, w_ref, b_ref, ln_ref,
                 q_out, k_out, v_out, r1_out):
    x = hid_ref[...] + res_ref[...]
    r1_out[...] = x
    ms = jnp.mean(x * x, axis=1, keepdims=True)
    h = x * jax.lax.rsqrt(ms + EPS) * ln_ref[...]
    qkv = jax.lax.dot_general(
        h.astype(jnp.bfloat16), w_ref[...], _NT,
        preferred_element_type=jnp.float32) + b_ref[...]

    pos = pos_ref[...]
    jq = jax.lax.rem(jax.lax.broadcasted_iota(jnp.int32, (1, NH * HALF), 1),
                     HALF).astype(jnp.float32)
    inv_freq = jnp.exp(jq * (-math.log(THETA) / HALF))
    f = pos * inv_freq
    cos_q = jnp.cos(f)
    sin_q = jnp.sin(f)
    cos_k = cos_q[:, :NKV * HALF]
    sin_k = sin_q[:, :NKV * HALF]

    qh = _split_halves(qkv[:, :Q_SIZE], NH)
    kh = _split_halves(qkv[:, Q_SIZE:Q_SIZE + KV_SIZE], NKV)
    q1 = qh[:, :NH * HALF]
    q2 = qh[:, NH * HALF:]
    k1 = kh[:, :NKV * HALF]
    k2 = kh[:, NKV * HALF:]
    v = qkv[:, Q_SIZE + KV_SIZE:]

    scale = HD ** -0.5
    q_out[...] = (jnp.concatenate(
        [q1 * cos_q - q2 * sin_q, q2 * cos_q + q1 * sin_q],
        axis=1) * scale).astype(jnp.bfloat16)
    k_out[...] = jnp.concatenate(
        [k1 * cos_k - k2 * sin_k, k2 * cos_k + k1 * sin_k],
        axis=1).astype(jnp.bfloat16)
    v_out[...] = v.astype(jnp.bfloat16)


def _stage2_body(q_ref, k_ref, v_ref, r1_ref, wo_ref, bo_ref, ln2_ref,
                 wr_ref, br_ref, wgu_ref, bgu_ref, wd_ref, bd_ref,
                 out_ref, r2_out):
    i = pl.program_id(0)
    q0 = i * BLK
    R = REP * BLK

    row = jax.lax.rem(jax.lax.broadcasted_iota(jnp.int32, (R, 1), 0), BLK)
    col = jax.lax.broadcasted_iota(jnp.int32, (1, S), 1)
    mask = col <= (q0 + row)

    o_cols = []
    for g in range(NKV):
        qs = []
        for hh in range(REP):
            h = g * REP + hh
            qs.append(jnp.concatenate(
                [q_ref[:, h * HALF:(h + 1) * HALF],
                 q_ref[:, NH * HALF + h * HALF:NH * HALF + (h + 1) * HALF]],
                axis=1))
        q_g = jnp.concatenate(qs, axis=0)

        k_g = jnp.concatenate(
            [k_ref[:, g * HALF:(g + 1) * HALF],
             k_ref[:, NKV * HALF + g * HALF:NKV * HALF + (g + 1) * HALF]],
            axis=1)
        v_g = v_ref[:, g * HD:(g + 1) * HD]
        s = jax.lax.dot_general(q_g, k_g, _NT,
                                preferred_element_type=jnp.float32)
        s = jnp.where(mask, s, _NEG)
        m = jnp.max(s, axis=1, keepdims=True)
        p = jnp.exp(s - m)
        l = jnp.sum(p, axis=1, keepdims=True)
        o_g = jnp.dot(p.astype(jnp.bfloat16), v_g,
                      preferred_element_type=jnp.float32) / l
        for hh in range(REP):
            o_cols.append(o_g[hh * BLK:(hh + 1) * BLK, :])
    o = jnp.concatenate(o_cols, axis=1).astype(jnp.bfloat16)

    attn = jax.lax.dot_general(
        o, wo_ref[...], _NT, preferred_element_type=jnp.float32) + bo_ref[...]
    r2 = attn + r1_ref[...]
    r2_out[...] = r2

    ms = jnp.mean(r2 * r2, axis=1, keepdims=True)
    h2 = (r2 * jax.lax.rsqrt(ms + EPS) * ln2_ref[...]).astype(jnp.bfloat16)

    logits = jax.lax.dot_general(
        h2, wr_ref[...], _NT, preferred_element_type=jnp.float32) + br_ref[...]
    m1 = jnp.max(logits, axis=1, keepdims=True)
    s2 = jnp.max(jnp.where(logits >= m1, _NEG, logits), axis=1, keepdims=True)
    e2 = jnp.exp(s2 - m1)
    denom = 1.0 + e2
    factor = 1.0 / denom + e2 / denom

    gu = jax.lax.dot_general(
        h2, wgu_ref[...], _NT,
        preferred_element_type=jnp.float32) + bgu_ref[...]
    gate = gu[:, :I]
    up = gu[:, I:]
    x = gate * (up * jax.nn.sigmoid(up))
    eo = jax.lax.dot_general(
        x.astype(jnp.bfloat16), wd_ref[...], _NT,
        preferred_element_type=jnp.float32) + bd_ref[...]
    out_ref[...] = factor * eo


def kernel(positions, hidden_states, residual, w_qkv, b_qkv, w_o, b_o,
           w_router, b_router, w_gate_up, b_gate_up, w_down, b_down,
           ln1_w, ln2_w):
    f32 = jnp.float32
    bf16 = jnp.bfloat16
    pos = positions.astype(f32).reshape(S, 1)

    full = lambda shape: pl.BlockSpec(shape, lambda i: (0, 0))
    blk = lambda cols: pl.BlockSpec((BLK, cols), lambda i: (i, 0))

    q_ro, k_ro, v, r1 = pl.pallas_call(
        _stage1_body,
        grid=(GRID,),
        in_specs=[
            blk(1),
            blk(H),
            blk(H),
            full((Q_SIZE + 2 * KV_SIZE, H)),
            full((1, Q_SIZE + 2 * KV_SIZE)),
            full((1, H)),
        ],
        out_specs=[blk(Q_SIZE), blk(KV_SIZE), blk(KV_SIZE), blk(H)],
        out_shape=[
            jax.ShapeDtypeStruct((S, Q_SIZE), bf16),
            jax.ShapeDtypeStruct((S, KV_SIZE), bf16),
            jax.ShapeDtypeStruct((S, KV_SIZE), bf16),
            jax.ShapeDtypeStruct((S, H), f32),
        ],
    )(pos, hidden_states, residual, w_qkv.astype(bf16),
      b_qkv.reshape(1, -1).astype(f32), ln1_w.reshape(1, H).astype(f32))

    out, r2 = pl.pallas_call(
        _stage2_body,
        grid=(GRID,),
        in_specs=[
            blk(Q_SIZE),
            full((S, KV_SIZE)),
            full((S, KV_SIZE)),
            blk(H),
            full((H, Q_SIZE)),
            full((1, H)),
            full((1, H)),
            full((E, H)),
            full((1, E)),
            full((2 * I, H)),
            full((1, 2 * I)),
            full((H, I)),
            full((1, H)),
        ],
        out_specs=[blk(H), blk(H)],
        out_shape=[
            jax.ShapeDtypeStruct((S, H), f32),
            jax.ShapeDtypeStruct((S, H), f32),
        ],
    )(q_ro, k_ro, v, r1,
      w_o.astype(bf16), b_o.reshape(1, H).astype(f32),
      ln2_w.reshape(1, H).astype(f32),
      w_router.astype(bf16), b_router.reshape(1, E).astype(f32),
      w_gate_up.astype(bf16), b_gate_up.reshape(1, 2 * I).astype(f32),
      w_down.astype(bf16), b_down.reshape(1, H).astype(f32))

    return (out, r2)


# speedup vs baseline: 1.3279x; 1.0571x over previous
import math

import jax
import jax.numpy as jnp
from jax.experimental import pallas as pl
from jax.experimental.pallas import tpu as pltpu

S = 2048
H = 768
NH = 12
NKV = 4
HD = 64
HALF = HD // 2
I = 768
E = 64
THETA = 150000.0
EPS = 1e-6
BLK = 256
GRID = S // BLK
REP = NH // NKV
Q_SIZE = NH * HD
KV_SIZE = NKV * HD

_NEG = -1e30
_NT = (((1,), (1,)), ((), ()))


def _split_halves(x, nheads):
    h1 = [x[:, h * HD:h * HD + HALF] for h in range(nheads)]
    h2 = [x[:, h * HD + HALF:(h + 1) * HD] for h in range(nheads)]
    return jnp.concatenate(h1 + h2, axis=1)


def _stage1_body(pos_ref, hid_ref, res_ref, w_ref, b_ref, ln_ref,
                 q_out, k_out, v_out, r1_out, w_bf):
    @pl.when(pl.program_id(0) == 0)
    def _cast_weights():
        w_bf[...] = w_ref[...].astype(jnp.bfloat16)

    x = hid_ref[...] + res_ref[...]
    r1_out[...] = x
    ms = jnp.mean(x * x, axis=1, keepdims=True)
    h = x * jax.lax.rsqrt(ms + EPS) * ln_ref[...]
    qkv = jax.lax.dot_general(
        h.astype(jnp.bfloat16), w_bf[...], _NT,
        preferred_element_type=jnp.float32) + b_ref[...]

    pos = pos_ref[...]
    jq = jax.lax.rem(jax.lax.broadcasted_iota(jnp.int32, (1, NH * HALF), 1),
                     HALF).astype(jnp.float32)
    inv_freq = jnp.exp(jq * (-math.log(THETA) / HALF))
    f = pos * inv_freq
    cos_q = jnp.cos(f)
    sin_q = jnp.sin(f)
    cos_k = cos_q[:, :NKV * HALF]
    sin_k = sin_q[:, :NKV * HALF]

    qh = _split_halves(qkv[:, :Q_SIZE], NH)
    kh = _split_halves(qkv[:, Q_SIZE:Q_SIZE + KV_SIZE], NKV)
    q1 = qh[:, :NH * HALF]
    q2 = qh[:, NH * HALF:]
    k1 = kh[:, :NKV * HALF]
    k2 = kh[:, NKV * HALF:]
    v = qkv[:, Q_SIZE + KV_SIZE:]

    scale = HD ** -0.5 * math.log2(math.e)
    q_out[...] = (jnp.concatenate(
        [q1 * cos_q - q2 * sin_q, q2 * cos_q + q1 * sin_q],
        axis=1) * scale).astype(jnp.bfloat16)
    k_out[...] = jnp.concatenate(
        [k1 * cos_k - k2 * sin_k, k2 * cos_k + k1 * sin_k],
        axis=1).astype(jnp.bfloat16)
    v_out[...] = v.astype(jnp.bfloat16)


def _stage2_body(q_ref, k_ref, v_ref, r1_ref, wo_ref, bo_ref, ln2_ref,
                 wr_ref, br_ref, wgu_ref, bgu_ref, wd_ref, bd_ref,
                 out_ref, r2_out, wo_bf, wr_bf, wgu_bf, wd_bf):
    i = pl.program_id(0)

    @pl.when(i == 0)
    def _cast_weights():
        wo_bf[...] = wo_ref[...].astype(jnp.bfloat16)
        wr_bf[...] = wr_ref[...].astype(jnp.bfloat16)
        wgu_bf[...] = wgu_ref[...].astype(jnp.bfloat16)
        wd_bf[...] = wd_ref[...].astype(jnp.bfloat16)
    q0 = i * BLK
    R = REP * BLK

    row = jax.lax.rem(jax.lax.broadcasted_iota(jnp.int32, (R, 1), 0), BLK)
    col = jax.lax.broadcasted_iota(jnp.int32, (1, S), 1)
    mask = col <= (q0 + row)

    o_cols = []
    for g in range(NKV):
        qs = []
        for hh in range(REP):
            h = g * REP + hh
            qs.append(jnp.concatenate(
                [q_ref[:, h * HALF:(h + 1) * HALF],
                 q_ref[:, NH * HALF + h * HALF:NH * HALF + (h + 1) * HALF]],
                axis=1))
        q_g = jnp.concatenate(qs, axis=0)

        k_g = jnp.concatenate(
            [k_ref[:, g * HALF:(g + 1) * HALF],
             k_ref[:, NKV * HALF + g * HALF:NKV * HALF + (g + 1) * HALF]],
            axis=1)
        v_g = v_ref[:, g * HD:(g + 1) * HD]
        s = jax.lax.dot_general(q_g, k_g, _NT,
                                preferred_element_type=jnp.float32)
        s = jnp.where(mask, s, _NEG)
        m = jnp.max(s, axis=1, keepdims=True)
        p = jnp.exp2(s - m)
        l = jnp.sum(p, axis=1, keepdims=True)
        o_g = jnp.dot(p.astype(jnp.bfloat16), v_g,
                      preferred_element_type=jnp.float32) / l
        for hh in range(REP):
            o_cols.append(o_g[hh * BLK:(hh + 1) * BLK, :])
    o = jnp.concatenate(o_cols, axis=1).astype(jnp.bfloat16)

    attn = jax.lax.dot_general(
        o, wo_bf[...], _NT, preferred_element_type=jnp.float32) + bo_ref[...]
    r2 = attn + r1_ref[...]
    r2_out[...] = r2

    ms = jnp.mean(r2 * r2, axis=1, keepdims=True)
    h2 = (r2 * jax.lax.rsqrt(ms + EPS) * ln2_ref[...]).astype(jnp.bfloat16)

    logits = jax.lax.dot_general(
        h2, wr_bf[...], _NT, preferred_element_type=jnp.float32) + br_ref[...]
    m1 = jnp.max(logits, axis=1, keepdims=True)
    s2 = jnp.max(jnp.where(logits >= m1, _NEG, logits), axis=1, keepdims=True)
    e2 = jnp.exp(s2 - m1)
    denom = 1.0 + e2
    factor = 1.0 / denom + e2 / denom

    gu = jax.lax.dot_general(
        h2, wgu_bf[...], _NT,
        preferred_element_type=jnp.float32) + bgu_ref[...]
    gate = gu[:, :I]
    up = gu[:, I:]
    x = gate * (up * jax.nn.sigmoid(up))
    eo = jax.lax.dot_general(
        x.astype(jnp.bfloat16), wd_bf[...], _NT,
        preferred_element_type=jnp.float32) + bd_ref[...]
    out_ref[...] = factor * eo


def kernel(positions, hidden_states, residual, w_qkv, b_qkv, w_o, b_o,
           w_router, b_router, w_gate_up, b_gate_up, w_down, b_down,
           ln1_w, ln2_w):
    f32 = jnp.float32
    bf16 = jnp.bfloat16
    pos = positions.astype(f32).reshape(S, 1)

    full = lambda shape: pl.BlockSpec(shape, lambda i: (0, 0))
    blk = lambda cols: pl.BlockSpec((BLK, cols), lambda i: (i, 0))

    q_ro, k_ro, v, r1 = pl.pallas_call(
        _stage1_body,
        grid=(GRID,),
        in_specs=[
            blk(1),
            blk(H),
            blk(H),
            full((Q_SIZE + 2 * KV_SIZE, H)),
            full((1, Q_SIZE + 2 * KV_SIZE)),
            full((1, H)),
        ],
        out_specs=[blk(Q_SIZE), blk(KV_SIZE), blk(KV_SIZE), blk(H)],
        out_shape=[
            jax.ShapeDtypeStruct((S, Q_SIZE), bf16),
            jax.ShapeDtypeStruct((S, KV_SIZE), bf16),
            jax.ShapeDtypeStruct((S, KV_SIZE), bf16),
            jax.ShapeDtypeStruct((S, H), f32),
        ],
        scratch_shapes=[pltpu.VMEM((Q_SIZE + 2 * KV_SIZE, H), bf16)],
    )(pos, hidden_states, residual, w_qkv,
      b_qkv.reshape(1, -1).astype(f32), ln1_w.reshape(1, H).astype(f32))

    out, r2 = pl.pallas_call(
        _stage2_body,
        grid=(GRID,),
        in_specs=[
            blk(Q_SIZE),
            full((S, KV_SIZE)),
            full((S, KV_SIZE)),
            blk(H),
            full((H, Q_SIZE)),
            full((1, H)),
            full((1, H)),
            full((E, H)),
            full((1, E)),
            full((2 * I, H)),
            full((1, 2 * I)),
            full((H, I)),
            full((1, H)),
        ],
        out_specs=[blk(H), blk(H)],
        out_shape=[
            jax.ShapeDtypeStruct((S, H), f32),
            jax.ShapeDtypeStruct((S, H), f32),
        ],
        scratch_shapes=[
            pltpu.VMEM((H, Q_SIZE), bf16),
            pltpu.VMEM((E, H), bf16),
            pltpu.VMEM((2 * I, H), bf16),
            pltpu.VMEM((H, I), bf16),
        ],
    )(q_ro, k_ro, v, r1,
      w_o, b_o.reshape(1, H).astype(f32),
      ln2_w.reshape(1, H).astype(f32),
      w_router, b_router.reshape(1, E).astype(f32),
      w_gate_up, b_gate_up.reshape(1, 2 * I).astype(f32),
      w_down, b_down.reshape(1, H).astype(f32))

    return (out, r2)
